# baseline (speedup 1.0000x reference)
import functools

import jax
import jax.numpy as jnp
from jax import lax
from jax.experimental import pallas as pl
from jax.experimental.pallas import tpu as pltpu
from jax.experimental.pallas import tpu_sc as plsc

N_NODES = 10000
N_EDGES = 320000
D = 128

NC = 2
NS = 16
NW = NC * NS

EDGES_PER_TILE = N_EDGES // NW
CHUNK = 80
N_CHUNKS = 128
PADDED = N_CHUNKS * CHUNK
GCH = 32
N_GROUPS = N_CHUNKS // GCH
ACC_ROWS = N_NODES + 8

ROW_CHUNK = 80
N_ROW_CHUNKS = N_NODES // ROW_CHUNK
ROW_CHUNKS_PER_TILE = -(-N_ROW_CHUNKS // NS)

ROW_BLK = 1000
N_BLK = N_NODES // ROW_BLK


def _mm_body(x_ref, w1_ref, w2_ref, y_ref):
    h = jax.lax.dot_general(x_ref[...], w1_ref[...], (((1,), (1,)), ((), ())),
                            precision=lax.Precision.HIGHEST,
                            preferred_element_type=jnp.float32)
    y_ref[...] = jax.lax.dot_general(h, w2_ref[...], (((1,), (1,)), ((), ())),
                                     precision=lax.Precision.HIGHEST,
                                     preferred_element_type=jnp.float32)


def _combine_body(p_ref, bias_ref, w2_ref, b2_ref, out_ref):
    c = jax.lax.dot_general(bias_ref[...], w2_ref[...], (((1,), (1,)), ((), ())),
                            precision=lax.Precision.HIGHEST,
                            preferred_element_type=jnp.float32) + b2_ref[...]
    out_ref[...] = p_ref[0] + p_ref[1] + c


def _sc_body(src_hbm, dst_hbm, y_hbm, out_hbm, acc, idx_s, idx_d, rows,
             sem_g, sem_s, sem_i):
    cc = lax.axis_index("c")
    ss = lax.axis_index("s")
    wid = cc * NS + ss

    def zero_row(i, _):
        for j in range(D // 16):
            rows[0, i, pl.ds(j * 16, 16)] = jnp.zeros((16,), jnp.float32)
        return _
    lax.fori_loop(0, ROW_CHUNK, zero_row, None)
    for k in range(ROW_CHUNKS_PER_TILE):
        cid = ss * ROW_CHUNKS_PER_TILE + k

        @pl.when(cid < N_ROW_CHUNKS)
        def _():
            r0 = pl.multiple_of(cid * ROW_CHUNK, ROW_CHUNK)
            pltpu.sync_copy(rows.at[0], acc.at[pl.ds(r0, ROW_CHUNK), :])

    pltpu.sync_copy(src_hbm.at[wid, pl.ds(0, GCH)], idx_s.at[0])
    pltpu.sync_copy(dst_hbm.at[wid, pl.ds(0, GCH)], idx_d.at[0])
    pltpu.async_copy(y_hbm.at[idx_s.at[0, 0]], rows.at[0], sem_g)
    plsc.subcore_barrier()

    def _wait_row(sem, b):
        pltpu.make_async_copy(y_hbm.at[pl.ds(0, CHUNK)], rows.at[b], sem).wait()

    def _idx_group(gn):
        p = gn % 2
        o = gn * GCH
        return [(src_hbm.at[wid, pl.ds(o, GCH)], idx_s.at[p]),
                (dst_hbm.at[wid, pl.ds(o, GCH)], idx_d.at[p])]

    def step(b, dst_row, gather_row, guard_first=None):
        bn = 1 - b
        if guard_first is None:
            _wait_row(sem_s, bn)
        else:
            @pl.when(guard_first)
            def _():
                _wait_row(sem_s, bn)
        if gather_row is not None:
            pltpu.async_copy(y_hbm.at[gather_row], rows.at[bn], sem_g)
        _wait_row(sem_g, b)
        pltpu.async_copy(rows.at[b], acc.at[dst_row], sem_s, add=True)

    for g in range(N_GROUPS):
        p = g % 2
        if g + 1 < N_GROUPS:
            for s_, d_ in _idx_group(g + 1):
                pltpu.async_copy(s_, d_, sem_i)

        def pair(k, _, g=g, p=p):
            for b in range(2):
                tl = 2 * k + b
                step(b, idx_d.at[p, tl], idx_s.at[p, tl + 1],
                     guard_first=(tl >= 1) if g == 0 else None)
            return _
        lax.fori_loop(0, GCH // 2 - 1, pair, None)

        step(0, idx_d.at[p, GCH - 2], idx_s.at[p, GCH - 1])
        if g + 1 < N_GROUPS:
            for s_, d_ in _idx_group(g + 1):
                pltpu.make_async_copy(s_, d_, sem_i).wait()
            step(1, idx_d.at[p, GCH - 1], idx_s.at[1 - p, 0])
        else:
            step(1, idx_d.at[p, GCH - 1], None)
    _wait_row(sem_s, 0)
    plsc.subcore_barrier()

    for k in range(ROW_CHUNKS_PER_TILE):
        cid = ss * ROW_CHUNKS_PER_TILE + k

        @pl.when(cid < N_ROW_CHUNKS)
        def _():
            r0 = pl.multiple_of(cid * ROW_CHUNK, ROW_CHUNK)
            pltpu.sync_copy(acc.at[pl.ds(r0, ROW_CHUNK), :], rows.at[0])
            pltpu.sync_copy(rows.at[0], out_hbm.at[cc, pl.ds(r0, ROW_CHUNK), :])


def _sc_propagate(src, dst, y):
    mesh = plsc.VectorSubcoreMesh(core_axis_name="c", subcore_axis_name="s",
                                  num_cores=NC, num_subcores=NS)
    f = pl.kernel(
        _sc_body,
        out_type=jax.ShapeDtypeStruct((NC, N_NODES, D), jnp.float32),
        mesh=mesh,
        scratch_types=[
            pltpu.VMEM_SHARED((ACC_ROWS, D), jnp.float32),
            pltpu.VMEM((2, GCH, CHUNK), jnp.int32),
            pltpu.VMEM((2, GCH, CHUNK), jnp.int32),
            pltpu.VMEM((2, CHUNK, D), jnp.float32),
            pltpu.SemaphoreType.DMA,
            pltpu.SemaphoreType.DMA,
            pltpu.SemaphoreType.DMA,
        ],
    )
    return f(src, dst, y)


def kernel(x, edge_index, W1, bias, W2, b2):
    pad = PADDED - EDGES_PER_TILE
    src = jnp.pad(edge_index[0].reshape(NW, EDGES_PER_TILE),
                  ((0, 0), (0, pad))).reshape(NW, N_CHUNKS, CHUNK)
    dst = jnp.pad(edge_index[1].reshape(NW, EDGES_PER_TILE),
                  ((0, 0), (0, pad)),
                  constant_values=N_NODES).reshape(NW, N_CHUNKS, CHUNK)

    y = pl.pallas_call(
        _mm_body,
        grid=(N_BLK,),
        in_specs=[
            pl.BlockSpec((ROW_BLK, D), lambda i: (i, 0)),
            pl.BlockSpec((D, D), lambda i: (0, 0)),
            pl.BlockSpec((D, D), lambda i: (0, 0)),
        ],
        out_specs=pl.BlockSpec((ROW_BLK, D), lambda i: (i, 0)),
        out_shape=jax.ShapeDtypeStruct((N_NODES, D), jnp.float32),
    )(x, W1, W2)

    p = _sc_propagate(src, dst, y)

    out = pl.pallas_call(
        _combine_body,
        grid=(N_BLK,),
        in_specs=[
            pl.BlockSpec((NC, ROW_BLK, D), lambda i: (0, i, 0)),
            pl.BlockSpec((1, D), lambda i: (0, 0)),
            pl.BlockSpec((D, D), lambda i: (0, 0)),
            pl.BlockSpec((1, D), lambda i: (0, 0)),
        ],
        out_specs=pl.BlockSpec((ROW_BLK, D), lambda i: (i, 0)),
        out_shape=jax.ShapeDtypeStruct((N_NODES, D), jnp.float32),
    )(p, bias[None, :], W2, b2[None, :])

    return out
